# baseline (speedup 1.0000x reference)
import functools

import jax
import jax.numpy as jnp
from jax import lax
from jax.experimental import pallas as pl
from jax.experimental.pallas import tpu as pltpu
from jax.experimental.pallas import tpu_sc as plsc

_L = 16
_CHUNK = 4096


def _edge_loss_partials(packed, bs, VC, EP, num_cores, num_subcores):
    NW = num_cores * num_subcores
    bpw = bs // NW
    n_chunks = EP // _CHUNK

    mesh = plsc.VectorSubcoreMesh(core_axis_name="c", subcore_axis_name="s")

    @functools.partial(
        pl.kernel,
        mesh=mesh,
        compiler_params=pltpu.CompilerParams(
            needs_layout_passes=False, use_tc_tiling_on_sc=False
        ),
        out_type=jax.ShapeDtypeStruct((NW, _L), jnp.float32),
        scratch_types=[
            pltpu.VMEM((bpw * VC,), jnp.int32),
            pltpu.VMEM((2, _CHUNK), jnp.int32),
            pltpu.VMEM((2, _CHUNK), jnp.int32),
            pltpu.VMEM((_L,), jnp.float32),
            pltpu.SemaphoreType.DMA,
            pltpu.SemaphoreType.DMA,
        ],
    )
    def edge_loss_sc(
        p_hbm, out_hbm, vert_v, i0_v, i1_v, acc_v, sem0, sem1
    ):
        wid = lax.axis_index("s") * num_cores + lax.axis_index("c")
        base = wid * bpw
        sems = (sem0, sem1)

        def issue(k):
            p = k % 2
            h0 = pltpu.async_copy(
                p_hbm.at[bs, pl.ds(k * _CHUNK, _CHUNK)], i0_v.at[p], sems[p]
            )
            h1 = pltpu.async_copy(
                p_hbm.at[bs + 1, pl.ds(k * _CHUNK, _CHUNK)], i1_v.at[p], sems[p]
            )
            return h0, h1

        pending = {0: issue(0)}
        for b in range(bpw):
            pltpu.sync_copy(
                p_hbm.at[base + b], vert_v.at[pl.ds(b * VC, VC)]
            )

        n_acc = bpw * 3
        accs = [jnp.zeros((_L,), jnp.float32)] * n_acc
        for k in range(n_chunks):
            p = k % 2
            h0, h1 = pending.pop(k)
            h0.wait()
            h1.wait()
            if k + 1 < n_chunks:
                pending[k + 1] = issue(k + 1)

            def body(j, accs):
                accs = list(accs)
                i0 = i0_v[p, pl.ds(j * _L, _L)] * 3
                i1 = i1_v[p, pl.ds(j * _L, _L)] * 3
                for b in range(bpw):
                    for c in range(3):
                        off = b * VC + c
                        a = plsc.bitcast(
                            plsc.load_gather(vert_v, [i0 + off]), jnp.float32
                        )
                        bb = plsc.bitcast(
                            plsc.load_gather(vert_v, [i1 + off]), jnp.float32
                        )
                        d = a - bb
                        accs[b * 3 + c] = accs[b * 3 + c] + d * d
                return tuple(accs)

            accs = lax.fori_loop(
                0, _CHUNK // _L, body, tuple(accs), unroll=2
            )

        total = accs[0]
        for a in accs[1:]:
            total = total + a
        acc_v[...] = total
        pltpu.sync_copy(acc_v, out_hbm.at[wid])

    return edge_loss_sc(packed)


def kernel(vertices, v0, v1):
    bs, V, C = vertices.shape
    VC = V * C
    E = v0.shape[0]
    info = plsc.get_sparse_core_info()
    EP = ((E + _CHUNK - 1) // _CHUNK) * _CHUNK
    assert EP <= VC
    v0p = jnp.pad(v0.astype(jnp.int32), (0, VC - E))
    v1p = jnp.pad(v1.astype(jnp.int32), (0, VC - E))
    packed = jnp.concatenate(
        [
            jax.lax.bitcast_convert_type(vertices.reshape(bs, VC), jnp.int32),
            v0p[None],
            v1p[None],
        ],
        axis=0,
    )
    partials = _edge_loss_partials(
        packed, bs, VC, EP, info.num_cores, info.num_subcores
    )
    return (partials.sum() / (E * bs)).astype(jnp.float32)


# speedup vs baseline: 1.1085x; 1.1085x over previous
import functools

import jax
import jax.numpy as jnp
from jax import lax
from jax.experimental import pallas as pl
from jax.experimental.pallas import tpu as pltpu
from jax.experimental.pallas import tpu_sc as plsc

_L = 16
_CHUNK = 4096


def _edge_loss_partials(vflat, v0p, v1p, num_cores, num_subcores):
    bs, VC = vflat.shape
    EP = v0p.shape[0]
    NW = num_cores * num_subcores
    bpw = bs // NW
    n_chunks = EP // _CHUNK

    mesh = plsc.VectorSubcoreMesh(core_axis_name="c", subcore_axis_name="s")

    @functools.partial(
        pl.kernel,
        mesh=mesh,
        compiler_params=pltpu.CompilerParams(
            needs_layout_passes=False, use_tc_tiling_on_sc=False
        ),
        out_type=jax.ShapeDtypeStruct((NW, _L), jnp.float32),
        scratch_types=[
            pltpu.VMEM((bpw * VC,), jnp.float32),
            pltpu.VMEM((2, _CHUNK), jnp.int32),
            pltpu.VMEM((2, _CHUNK), jnp.int32),
            pltpu.VMEM((_L,), jnp.float32),
            pltpu.SemaphoreType.DMA,
            pltpu.SemaphoreType.DMA,
        ],
    )
    def edge_loss_sc(
        v_hbm, v0_hbm, v1_hbm, out_hbm, vert_v, i0_v, i1_v, acc_v, sem0, sem1
    ):
        wid = lax.axis_index("s") * num_cores + lax.axis_index("c")
        base = wid * bpw
        sems = (sem0, sem1)

        def issue(k):
            p = k % 2
            h0 = pltpu.async_copy(
                v0_hbm.at[pl.ds(k * _CHUNK, _CHUNK)], i0_v.at[p], sems[p]
            )
            h1 = pltpu.async_copy(
                v1_hbm.at[pl.ds(k * _CHUNK, _CHUNK)], i1_v.at[p], sems[p]
            )
            return h0, h1

        pending = {0: issue(0)}
        for b in range(bpw):
            pltpu.sync_copy(
                v_hbm.at[base + b], vert_v.at[pl.ds(b * VC, VC)]
            )

        n_acc = bpw * 3
        accs = [jnp.zeros((_L,), jnp.float32)] * n_acc
        for k in range(n_chunks):
            p = k % 2
            h0, h1 = pending.pop(k)
            h0.wait()
            h1.wait()
            if k + 1 < n_chunks:
                pending[k + 1] = issue(k + 1)

            def body(j, accs):
                accs = list(accs)
                i0 = i0_v[p, pl.ds(j * _L, _L)] * 3
                i1 = i1_v[p, pl.ds(j * _L, _L)] * 3
                for b in range(bpw):
                    for c in range(3):
                        off = b * VC + c
                        a = plsc.load_gather(vert_v, [i0 + off])
                        bb = plsc.load_gather(vert_v, [i1 + off])
                        d = a - bb
                        accs[b * 3 + c] = accs[b * 3 + c] + d * d
                return tuple(accs)

            accs = lax.fori_loop(
                0, _CHUNK // _L, body, tuple(accs), unroll=2
            )

        total = accs[0]
        for a in accs[1:]:
            total = total + a
        acc_v[...] = total
        pltpu.sync_copy(acc_v, out_hbm.at[wid])

    return edge_loss_sc(vflat, v0p, v1p)


def kernel(vertices, v0, v1):
    bs, V, C = vertices.shape
    E = v0.shape[0]
    info = plsc.get_sparse_core_info()
    EP = ((E + _CHUNK - 1) // _CHUNK) * _CHUNK
    v0p = jnp.pad(v0.astype(jnp.int32), (0, EP - E))
    v1p = jnp.pad(v1.astype(jnp.int32), (0, EP - E))
    partials = _edge_loss_partials(
        vertices.reshape(bs, V * C), v0p, v1p,
        info.num_cores, info.num_subcores,
    )
    return (partials.sum() / (E * bs)).astype(jnp.float32)


# speedup vs baseline: 1.1813x; 1.0657x over previous
import functools

import jax
import jax.numpy as jnp
from jax import lax
from jax.experimental import pallas as pl
from jax.experimental.pallas import tpu as pltpu
from jax.experimental.pallas import tpu_sc as plsc

_L = 16
_CHUNK = 4096


def _edge_loss_partials(vflat, v0p, v1p, num_cores, num_subcores):
    bs, VC = vflat.shape
    EP = v0p.shape[0]
    NW = num_cores * num_subcores
    bpw = bs // NW
    n_chunks = EP // _CHUNK

    mesh = plsc.VectorSubcoreMesh(core_axis_name="c", subcore_axis_name="s")

    @functools.partial(
        pl.kernel,
        mesh=mesh,
        compiler_params=pltpu.CompilerParams(
            needs_layout_passes=False, use_tc_tiling_on_sc=True
        ),
        out_type=jax.ShapeDtypeStruct((NW, _L), jnp.float32),
        scratch_types=[
            pltpu.VMEM((bpw * VC,), jnp.float32),
            pltpu.VMEM((2, _CHUNK), jnp.int32),
            pltpu.VMEM((2, _CHUNK), jnp.int32),
            pltpu.VMEM((_L,), jnp.float32),
            pltpu.SemaphoreType.DMA,
            pltpu.SemaphoreType.DMA,
        ],
    )
    def edge_loss_sc(
        v_hbm, v0_hbm, v1_hbm, out_hbm, vert_v, i0_v, i1_v, acc_v, sem0, sem1
    ):
        wid = lax.axis_index("s") * num_cores + lax.axis_index("c")
        base = wid * bpw
        sems = (sem0, sem1)

        def issue(k):
            p = k % 2
            h0 = pltpu.async_copy(
                v0_hbm.at[pl.ds(k * _CHUNK, _CHUNK)], i0_v.at[p], sems[p]
            )
            h1 = pltpu.async_copy(
                v1_hbm.at[pl.ds(k * _CHUNK, _CHUNK)], i1_v.at[p], sems[p]
            )
            return h0, h1

        pending = {0: issue(0)}
        for b in range(bpw):
            pltpu.sync_copy(
                v_hbm.at[base + b], vert_v.at[pl.ds(b * VC, VC)]
            )

        n_acc = bpw * 3
        accs = [jnp.zeros((_L,), jnp.float32)] * n_acc
        for k in range(n_chunks):
            p = k % 2
            h0, h1 = pending.pop(k)
            h0.wait()
            h1.wait()
            if k + 1 < n_chunks:
                pending[k + 1] = issue(k + 1)

            def body(j, accs):
                accs = list(accs)
                i0 = i0_v[p, pl.ds(j * _L, _L)] * 3
                i1 = i1_v[p, pl.ds(j * _L, _L)] * 3
                for b in range(bpw):
                    for c in range(3):
                        off = b * VC + c
                        a = plsc.load_gather(vert_v, [i0 + off])
                        bb = plsc.load_gather(vert_v, [i1 + off])
                        d = a - bb
                        accs[b * 3 + c] = accs[b * 3 + c] + d * d
                return tuple(accs)

            accs = lax.fori_loop(
                0, _CHUNK // _L, body, tuple(accs), unroll=2
            )

        total = accs[0]
        for a in accs[1:]:
            total = total + a
        acc_v[...] = total
        pltpu.sync_copy(acc_v, out_hbm.at[wid])

    return edge_loss_sc(vflat, v0p, v1p)


def kernel(vertices, v0, v1):
    bs, V, C = vertices.shape
    E = v0.shape[0]
    info = plsc.get_sparse_core_info()
    EP = ((E + _CHUNK - 1) // _CHUNK) * _CHUNK
    v0p = jnp.pad(v0.astype(jnp.int32), (0, EP - E))
    v1p = jnp.pad(v1.astype(jnp.int32), (0, EP - E))
    partials = _edge_loss_partials(
        vertices.reshape(bs, V * C), v0p, v1p,
        info.num_cores, info.num_subcores,
    )
    return (partials.sum() / (E * bs)).astype(jnp.float32)


# speedup vs baseline: 2.0400x; 1.7268x over previous
import functools

import jax
import jax.numpy as jnp
from jax import lax
from jax.experimental import pallas as pl
from jax.experimental.pallas import tpu as pltpu
from jax.experimental.pallas import tpu_sc as plsc

_L = 16
_CHUNK = 4096


def _edge_loss_partials(vplanes, v0p, v1p, num_cores, num_subcores):
    C, bs, V = vplanes.shape
    EP = v0p.shape[0]
    NW = num_cores * num_subcores
    bpw = bs // NW
    n_chunks = EP // _CHUNK

    mesh = plsc.VectorSubcoreMesh(core_axis_name="c", subcore_axis_name="s")

    @functools.partial(
        pl.kernel,
        mesh=mesh,
        compiler_params=pltpu.CompilerParams(
            needs_layout_passes=False, use_tc_tiling_on_sc=True
        ),
        out_type=jax.ShapeDtypeStruct((NW, _L), jnp.float32),
        scratch_types=[
            pltpu.VMEM((bpw * C * V,), jnp.float32),
            pltpu.VMEM((2, _CHUNK), jnp.int32),
            pltpu.VMEM((2, _CHUNK), jnp.int32),
            pltpu.VMEM((_L,), jnp.float32),
            pltpu.SemaphoreType.DMA,
            pltpu.SemaphoreType.DMA,
        ],
    )
    def edge_loss_sc(
        v_hbm, v0_hbm, v1_hbm, out_hbm, vert_v, i0_v, i1_v, acc_v, sem0, sem1
    ):
        wid = lax.axis_index("s") * num_cores + lax.axis_index("c")
        base = wid * bpw
        sems = (sem0, sem1)

        def issue(k):
            p = k % 2
            h0 = pltpu.async_copy(
                v0_hbm.at[pl.ds(k * _CHUNK, _CHUNK)], i0_v.at[p], sems[p]
            )
            h1 = pltpu.async_copy(
                v1_hbm.at[pl.ds(k * _CHUNK, _CHUNK)], i1_v.at[p], sems[p]
            )
            return h0, h1

        pending = {0: issue(0)}
        for b in range(bpw):
            for c in range(C):
                pltpu.sync_copy(
                    v_hbm.at[c, base + b],
                    vert_v.at[pl.ds((b * C + c) * V, V)],
                )

        n_acc = bpw * 3
        accs = [jnp.zeros((_L,), jnp.float32)] * n_acc
        for k in range(n_chunks):
            p = k % 2
            h0, h1 = pending.pop(k)
            h0.wait()
            h1.wait()
            if k + 1 < n_chunks:
                pending[k + 1] = issue(k + 1)

            def body(j, accs):
                accs = list(accs)
                i0 = i0_v[p, pl.ds(j * _L, _L)]
                i1 = i1_v[p, pl.ds(j * _L, _L)]
                for b in range(bpw):
                    for c in range(3):
                        off = (b * C + c) * V
                        a = plsc.load_gather(vert_v, [i0 + off])
                        bb = plsc.load_gather(vert_v, [i1 + off])
                        d = a - bb
                        accs[b * 3 + c] = accs[b * 3 + c] + d * d
                return tuple(accs)

            accs = lax.fori_loop(
                0, _CHUNK // _L, body, tuple(accs), unroll=2
            )

        total = accs[0]
        for a in accs[1:]:
            total = total + a
        acc_v[...] = total
        pltpu.sync_copy(acc_v, out_hbm.at[wid])

    return edge_loss_sc(vplanes, v0p, v1p)


def kernel(vertices, v0, v1):
    bs, V, C = vertices.shape
    E = v0.shape[0]
    info = plsc.get_sparse_core_info()
    EP = ((E + _CHUNK - 1) // _CHUNK) * _CHUNK
    v0p = jnp.pad(v0.astype(jnp.int32), (0, EP - E))
    v1p = jnp.pad(v1.astype(jnp.int32), (0, EP - E))
    partials = _edge_loss_partials(
        jnp.moveaxis(vertices, 2, 0), v0p, v1p,
        info.num_cores, info.num_subcores,
    )
    return (partials.sum() / (E * bs)).astype(jnp.float32)


# speedup vs baseline: 2.0908x; 1.0249x over previous
import functools

import jax
import jax.numpy as jnp
from jax import lax
from jax.experimental import pallas as pl
from jax.experimental.pallas import tpu as pltpu
from jax.experimental.pallas import tpu_sc as plsc

_L = 16
_CHUNK = 6144


def _edge_loss_partials(vplanes, v0p, v1p, num_cores, num_subcores):
    C, bs, V = vplanes.shape
    EP = v0p.shape[0]
    NW = num_cores * num_subcores
    bpw = bs // NW
    n_chunks = EP // _CHUNK

    mesh = plsc.VectorSubcoreMesh(core_axis_name="c", subcore_axis_name="s")

    @functools.partial(
        pl.kernel,
        mesh=mesh,
        compiler_params=pltpu.CompilerParams(
            needs_layout_passes=False, use_tc_tiling_on_sc=True
        ),
        out_type=jax.ShapeDtypeStruct((NW, _L), jnp.float32),
        scratch_types=[
            pltpu.VMEM((bpw * C * V,), jnp.float32),
            pltpu.VMEM((2, _CHUNK), jnp.int32),
            pltpu.VMEM((2, _CHUNK), jnp.int32),
            pltpu.VMEM((_L,), jnp.float32),
            pltpu.SemaphoreType.DMA,
            pltpu.SemaphoreType.DMA,
            pltpu.SemaphoreType.DMA,
        ],
    )
    def edge_loss_sc(
        v_hbm, v0_hbm, v1_hbm, out_hbm,
        vert_v, i0_v, i1_v, acc_v, sem0, sem1, slab_sem
    ):
        wid = lax.axis_index("s") * num_cores + lax.axis_index("c")
        base = wid * bpw
        sems = (sem0, sem1)
        n_steps = bpw * n_chunks

        def issue(s):
            k = s % n_chunks
            p = s % 2
            h0 = pltpu.async_copy(
                v0_hbm.at[pl.ds(k * _CHUNK, _CHUNK)], i0_v.at[p], sems[p]
            )
            h1 = pltpu.async_copy(
                v1_hbm.at[pl.ds(k * _CHUNK, _CHUNK)], i1_v.at[p], sems[p]
            )
            return h0, h1

        pending = {0: issue(0)}
        slabs = []
        for b in range(bpw):
            slabs.append([
                pltpu.async_copy(
                    v_hbm.at[c, base + b],
                    vert_v.at[pl.ds((b * C + c) * V, V)],
                    slab_sem,
                )
                for c in range(C)
            ])

        accs = [jnp.zeros((_L,), jnp.float32)] * (bpw * 3)
        for b in range(bpw):
            for h in slabs[b]:
                h.wait()
            for k in range(n_chunks):
                s = b * n_chunks + k
                p = s % 2
                h0, h1 = pending.pop(s)
                h0.wait()
                h1.wait()
                if s + 1 < n_steps:
                    pending[s + 1] = issue(s + 1)

                def body(j, accs):
                    accs = list(accs)
                    i0 = i0_v[p, pl.ds(j * _L, _L)]
                    i1 = i1_v[p, pl.ds(j * _L, _L)]
                    for c in range(3):
                        off = (b * C + c) * V
                        a = plsc.load_gather(vert_v, [i0 + off])
                        bb = plsc.load_gather(vert_v, [i1 + off])
                        d = a - bb
                        accs[b * 3 + c] = accs[b * 3 + c] + d * d
                    return tuple(accs)

                accs = lax.fori_loop(
                    0, _CHUNK // _L, body, tuple(accs), unroll=4
                )

        total = accs[0]
        for a in accs[1:]:
            total = total + a
        acc_v[...] = total
        pltpu.sync_copy(acc_v, out_hbm.at[wid])

    return edge_loss_sc(vplanes, v0p, v1p)


def kernel(vertices, v0, v1):
    bs, V, C = vertices.shape
    E = v0.shape[0]
    info = plsc.get_sparse_core_info()
    EP = ((E + _CHUNK - 1) // _CHUNK) * _CHUNK
    v0p = jnp.pad(v0.astype(jnp.int32), (0, EP - E))
    v1p = jnp.pad(v1.astype(jnp.int32), (0, EP - E))
    partials = _edge_loss_partials(
        jnp.moveaxis(vertices, 2, 0), v0p, v1p,
        info.num_cores, info.num_subcores,
    )
    return (partials.sum() / (E * bs)).astype(jnp.float32)


# speedup vs baseline: 2.2065x; 1.0553x over previous
import functools

import jax
import jax.numpy as jnp
from jax import lax
from jax.experimental import pallas as pl
from jax.experimental.pallas import tpu as pltpu
from jax.experimental.pallas import tpu_sc as plsc

_L = 16
_CHUNK = 6144


def _edge_loss_partials(vplanes, v0p, v1p, num_cores, num_subcores):
    C, bs, V = vplanes.shape
    EP = v0p.shape[0]
    NW = num_cores * num_subcores
    bpw = bs // NW
    n_chunks = EP // _CHUNK

    mesh = plsc.VectorSubcoreMesh(core_axis_name="c", subcore_axis_name="s")

    @functools.partial(
        pl.kernel,
        mesh=mesh,
        compiler_params=pltpu.CompilerParams(
            needs_layout_passes=False, use_tc_tiling_on_sc=True
        ),
        out_type=jax.ShapeDtypeStruct((NW, _L), jnp.float32),
        scratch_types=[
            pltpu.VMEM((bpw * C * V,), jnp.float32),
            pltpu.VMEM((2, _CHUNK), jnp.int32),
            pltpu.VMEM((2, _CHUNK), jnp.int32),
            pltpu.VMEM((_L,), jnp.float32),
            pltpu.SemaphoreType.DMA,
            pltpu.SemaphoreType.DMA,
            pltpu.SemaphoreType.DMA,
        ],
    )
    def edge_loss_sc(
        v_hbm, v0_hbm, v1_hbm, out_hbm,
        vert_v, i0_v, i1_v, acc_v, sem0, sem1, slab_sem
    ):
        wid = lax.axis_index("s") * num_cores + lax.axis_index("c")
        base = wid * bpw
        sems = (sem0, sem1)

        split = n_chunks // 2
        if bpw == 2:
            sched = (
                [(k, (0,)) for k in range(split)]
                + [(k, tuple(range(bpw))) for k in range(split, n_chunks)]
                + [(k, (1,)) for k in range(split)]
            )
        else:
            sched = [(k, (b,)) for b in range(bpw) for k in range(n_chunks)]
        n_steps = len(sched)

        def issue(s):
            k = sched[s][0]
            p = s % 2
            h0 = pltpu.async_copy(
                v0_hbm.at[pl.ds(k * _CHUNK, _CHUNK)], i0_v.at[p], sems[p]
            )
            h1 = pltpu.async_copy(
                v1_hbm.at[pl.ds(k * _CHUNK, _CHUNK)], i1_v.at[p], sems[p]
            )
            return h0, h1

        pending = {0: issue(0)}
        slabs = []
        for b in range(bpw):
            slabs.append([
                pltpu.async_copy(
                    v_hbm.at[c, base + b],
                    vert_v.at[pl.ds((b * C + c) * V, V)],
                    slab_sem,
                )
                for c in range(C)
            ])

        accs = [jnp.zeros((_L,), jnp.float32)] * (bpw * 3)
        waited = [False] * bpw
        for s in range(n_steps):
            k, bats = sched[s]
            p = s % 2
            for b in bats:
                if not waited[b]:
                    for h in slabs[b]:
                        h.wait()
                    waited[b] = True
            h0, h1 = pending.pop(s)
            h0.wait()
            h1.wait()
            if s + 1 < n_steps:
                pending[s + 1] = issue(s + 1)

            def body(j, accs):
                accs = list(accs)
                i0 = i0_v[p, pl.ds(j * _L, _L)]
                i1 = i1_v[p, pl.ds(j * _L, _L)]
                for b in bats:
                    for c in range(3):
                        off = (b * C + c) * V
                        a = plsc.load_gather(vert_v, [i0 + off])
                        bb = plsc.load_gather(vert_v, [i1 + off])
                        d = a - bb
                        accs[b * 3 + c] = accs[b * 3 + c] + d * d
                return tuple(accs)

            accs = lax.fori_loop(
                0, _CHUNK // _L, body, tuple(accs), unroll=4
            )

        total = accs[0]
        for a in accs[1:]:
            total = total + a
        acc_v[...] = total
        pltpu.sync_copy(acc_v, out_hbm.at[wid])

    return edge_loss_sc(vplanes, v0p, v1p)


def kernel(vertices, v0, v1):
    bs, V, C = vertices.shape
    E = v0.shape[0]
    info = plsc.get_sparse_core_info()
    EP = ((E + _CHUNK - 1) // _CHUNK) * _CHUNK
    v0p = jnp.pad(v0.astype(jnp.int32), (0, EP - E))
    v1p = jnp.pad(v1.astype(jnp.int32), (0, EP - E))
    partials = _edge_loss_partials(
        jnp.moveaxis(vertices, 2, 0), v0p, v1p,
        info.num_cores, info.num_subcores,
    )
    return (partials.sum() / (E * bs)).astype(jnp.float32)


# speedup vs baseline: 2.3365x; 1.0589x over previous
import functools

import jax
import jax.numpy as jnp
from jax import lax
from jax.experimental import pallas as pl
from jax.experimental.pallas import tpu as pltpu
from jax.experimental.pallas import tpu_sc as plsc

_L = 16
_CHUNK = 6144


def _edge_loss_partials(vplanes, v0, v1, num_cores, num_subcores):
    C, bs, V = vplanes.shape
    E = v0.shape[0]
    NW = num_cores * num_subcores
    bpw = bs // NW
    n_chunks = -(-E // _CHUNK)
    tail = E - (n_chunks - 1) * _CHUNK
    tail_vec = tail // _L
    tail_rem = tail - tail_vec * _L

    mesh = plsc.VectorSubcoreMesh(core_axis_name="c", subcore_axis_name="s")

    @functools.partial(
        pl.kernel,
        mesh=mesh,
        compiler_params=pltpu.CompilerParams(
            needs_layout_passes=False, use_tc_tiling_on_sc=True
        ),
        out_type=jax.ShapeDtypeStruct((NW, _L), jnp.float32),
        scratch_types=[
            pltpu.VMEM((bpw * C * V,), jnp.float32),
            pltpu.VMEM((_CHUNK,), jnp.int32),
            pltpu.VMEM((_CHUNK,), jnp.int32),
            pltpu.VMEM((_CHUNK,), jnp.int32),
            pltpu.VMEM((_CHUNK,), jnp.int32),
            pltpu.VMEM((_L,), jnp.float32),
            pltpu.SemaphoreType.DMA,
            pltpu.SemaphoreType.DMA,
            pltpu.SemaphoreType.DMA,
        ],
    )
    def edge_loss_sc(
        v_hbm, v0_hbm, v1_hbm, out_hbm,
        vert_v, i0a, i0b, i1a, i1b, acc_v, sem0, sem1, slab_sem
    ):
        i0_bufs = (i0a, i0b)
        i1_bufs = (i1a, i1b)
        wid = lax.axis_index("s") * num_cores + lax.axis_index("c")
        base = wid * bpw
        sems = (sem0, sem1)

        split = n_chunks // 2
        if bpw == 2:
            sched = (
                [(k, (0,)) for k in range(split)]
                + [(k, tuple(range(bpw))) for k in range(split, n_chunks)]
                + [(k, (1,)) for k in range(split)]
            )
        else:
            sched = [(k, (b,)) for b in range(bpw) for k in range(n_chunks)]
        n_steps = len(sched)

        def issue(s):
            k = sched[s][0]
            p = s % 2
            n = _CHUNK if k < n_chunks - 1 else tail
            h0 = pltpu.async_copy(
                v0_hbm.at[pl.ds(k * _CHUNK, n)],
                i0_bufs[p].at[pl.ds(0, n)],
                sems[p],
            )
            h1 = pltpu.async_copy(
                v1_hbm.at[pl.ds(k * _CHUNK, n)],
                i1_bufs[p].at[pl.ds(0, n)],
                sems[p],
            )
            return h0, h1

        pending = {0: issue(0)}
        slabs = []
        for b in range(bpw):
            slabs.append([
                pltpu.async_copy(
                    v_hbm.at[c, base + b],
                    vert_v.at[pl.ds((b * C + c) * V, V)],
                    slab_sem,
                )
                for c in range(C)
            ])

        accs = [jnp.zeros((_L,), jnp.float32)] * (bpw * 3)
        waited = [False] * bpw
        for s in range(n_steps):
            k, bats = sched[s]
            p = s % 2
            for b in bats:
                if not waited[b]:
                    for h in slabs[b]:
                        h.wait()
                    waited[b] = True
            h0, h1 = pending.pop(s)
            h0.wait()
            h1.wait()
            if s + 1 < n_steps:
                pending[s + 1] = issue(s + 1)

            def body(j, accs):
                accs = list(accs)
                i0 = i0_bufs[p][pl.ds(j * _L, _L)]
                i1 = i1_bufs[p][pl.ds(j * _L, _L)]
                for b in bats:
                    for c in range(3):
                        off = (b * C + c) * V
                        a = plsc.load_gather(vert_v, [i0 + off])
                        bb = plsc.load_gather(vert_v, [i1 + off])
                        d = a - bb
                        accs[b * 3 + c] = accs[b * 3 + c] + d * d
                return tuple(accs)

            is_tail = k == n_chunks - 1
            n_vec = tail_vec if is_tail else _CHUNK // _L
            accs = lax.fori_loop(0, n_vec, body, tuple(accs), unroll=4)

            if is_tail and tail_rem:
                accs = list(accs)
                msk = lax.iota(jnp.int32, _L) < tail_rem
                i0 = jnp.where(msk, i0_bufs[p][pl.ds(tail_vec * _L, _L)], 0)
                i1 = jnp.where(msk, i1_bufs[p][pl.ds(tail_vec * _L, _L)], 0)
                for b in bats:
                    for c in range(3):
                        off = (b * C + c) * V
                        a = plsc.load_gather(vert_v, [i0 + off])
                        bb = plsc.load_gather(vert_v, [i1 + off])
                        d = jnp.where(msk, a - bb, 0.0)
                        accs[b * 3 + c] = accs[b * 3 + c] + d * d
                accs = tuple(accs)

        total = accs[0]
        for a in accs[1:]:
            total = total + a
        acc_v[...] = total
        pltpu.sync_copy(acc_v, out_hbm.at[wid])

    return edge_loss_sc(vplanes, v0, v1)


def kernel(vertices, v0, v1):
    bs, V, C = vertices.shape
    E = v0.shape[0]
    info = plsc.get_sparse_core_info()
    partials = _edge_loss_partials(
        jnp.moveaxis(vertices, 2, 0),
        v0.astype(jnp.int32), v1.astype(jnp.int32),
        info.num_cores, info.num_subcores,
    )
    return (partials.sum() / (E * bs)).astype(jnp.float32)


# speedup vs baseline: 2.3736x; 1.0159x over previous
import functools

import jax
import jax.numpy as jnp
from jax import lax
from jax.experimental import pallas as pl
from jax.experimental.pallas import tpu as pltpu
from jax.experimental.pallas import tpu_sc as plsc

_L = 16
_CHUNK = 6144


def _edge_loss_partials(vplanes, v0, v1, num_cores, num_subcores):
    C, bs, V = vplanes.shape
    E = v0.shape[0]
    NW = num_cores * num_subcores
    bpw = bs // NW
    n_chunks = -(-E // _CHUNK)
    tail = E - (n_chunks - 1) * _CHUNK
    tail_vec = tail // _L
    tail_rem = tail - tail_vec * _L

    mesh = plsc.VectorSubcoreMesh(core_axis_name="c", subcore_axis_name="s")

    @functools.partial(
        pl.kernel,
        mesh=mesh,
        compiler_params=pltpu.CompilerParams(
            needs_layout_passes=False, use_tc_tiling_on_sc=True
        ),
        out_type=jax.ShapeDtypeStruct((NW, _L), jnp.float32),
        scratch_types=[
            pltpu.VMEM((bpw * C * V,), jnp.float32),
            pltpu.VMEM((_CHUNK,), jnp.int32),
            pltpu.VMEM((_CHUNK,), jnp.int32),
            pltpu.VMEM((_CHUNK,), jnp.int32),
            pltpu.VMEM((_CHUNK,), jnp.int32),
            pltpu.VMEM((_L,), jnp.float32),
            pltpu.SemaphoreType.DMA,
            pltpu.SemaphoreType.DMA,
            pltpu.SemaphoreType.DMA,
        ],
    )
    def edge_loss_sc(
        v_hbm, v0_hbm, v1_hbm, out_hbm,
        vert_v, i0a, i0b, i1a, i1b, acc_v, sem0, sem1, slab_sem
    ):
        i0_bufs = (i0a, i0b)
        i1_bufs = (i1a, i1b)
        wid = lax.axis_index("s") * num_cores + lax.axis_index("c")
        base = wid * bpw
        sems = (sem0, sem1)

        split = min(2, n_chunks // 2)
        if bpw == 2:
            sched = (
                [(k, (0,)) for k in range(split)]
                + [(k, tuple(range(bpw))) for k in range(split, n_chunks)]
                + [(k, (1,)) for k in range(split)]
            )
        else:
            sched = [(k, (b,)) for b in range(bpw) for k in range(n_chunks)]
        n_steps = len(sched)

        def issue(s):
            k = sched[s][0]
            p = s % 2
            n = _CHUNK if k < n_chunks - 1 else tail
            h0 = pltpu.async_copy(
                v0_hbm.at[pl.ds(k * _CHUNK, n)],
                i0_bufs[p].at[pl.ds(0, n)],
                sems[p],
            )
            h1 = pltpu.async_copy(
                v1_hbm.at[pl.ds(k * _CHUNK, n)],
                i1_bufs[p].at[pl.ds(0, n)],
                sems[p],
            )
            return h0, h1

        pending = {0: issue(0)}
        slabs = []
        for b in range(bpw):
            slabs.append([
                pltpu.async_copy(
                    v_hbm.at[c, base + b],
                    vert_v.at[pl.ds((b * C + c) * V, V)],
                    slab_sem,
                )
                for c in range(C)
            ])

        accs = [jnp.zeros((_L,), jnp.float32)] * (bpw * 3)
        waited = [False] * bpw
        for s in range(n_steps):
            k, bats = sched[s]
            p = s % 2
            for b in bats:
                if not waited[b]:
                    for h in slabs[b]:
                        h.wait()
                    waited[b] = True
            h0, h1 = pending.pop(s)
            h0.wait()
            h1.wait()
            if s + 1 < n_steps:
                pending[s + 1] = issue(s + 1)

            def body(j, accs):
                accs = list(accs)
                i0 = i0_bufs[p][pl.ds(j * _L, _L)]
                i1 = i1_bufs[p][pl.ds(j * _L, _L)]
                for b in bats:
                    for c in range(3):
                        off = (b * C + c) * V
                        a = plsc.load_gather(vert_v, [i0 + off])
                        bb = plsc.load_gather(vert_v, [i1 + off])
                        d = a - bb
                        accs[b * 3 + c] = accs[b * 3 + c] + d * d
                return tuple(accs)

            is_tail = k == n_chunks - 1
            n_vec = tail_vec if is_tail else _CHUNK // _L
            accs = lax.fori_loop(0, n_vec, body, tuple(accs), unroll=4)

            if is_tail and tail_rem:
                accs = list(accs)
                msk = lax.iota(jnp.int32, _L) < tail_rem
                i0 = jnp.where(msk, i0_bufs[p][pl.ds(tail_vec * _L, _L)], 0)
                i1 = jnp.where(msk, i1_bufs[p][pl.ds(tail_vec * _L, _L)], 0)
                for b in bats:
                    for c in range(3):
                        off = (b * C + c) * V
                        a = plsc.load_gather(vert_v, [i0 + off])
                        bb = plsc.load_gather(vert_v, [i1 + off])
                        d = jnp.where(msk, a - bb, 0.0)
                        accs[b * 3 + c] = accs[b * 3 + c] + d * d
                accs = tuple(accs)

        total = accs[0]
        for a in accs[1:]:
            total = total + a
        acc_v[...] = total
        pltpu.sync_copy(acc_v, out_hbm.at[wid])

    return edge_loss_sc(vplanes, v0, v1)


def kernel(vertices, v0, v1):
    bs, V, C = vertices.shape
    E = v0.shape[0]
    info = plsc.get_sparse_core_info()
    partials = _edge_loss_partials(
        jnp.moveaxis(vertices, 2, 0),
        v0.astype(jnp.int32), v1.astype(jnp.int32),
        info.num_cores, info.num_subcores,
    )
    return (partials.sum() / (E * bs)).astype(jnp.float32)
